# baseline (speedup 1.0000x reference)
import jax
import jax.numpy as jnp
from jax import lax
from jax.experimental import pallas as pl
from jax.experimental.pallas import tpu as pltpu
from jax.experimental.pallas import tpu_sc as plsc

N, E, D, H, B, L, C = 10000, 320000, 128, 128, 32, 3, 6
NC, NS = 2, 16
NW = NC * NS
EPT = E // NW
CH = 128
NCHUNK = 80
EPP = NCHUNK * CH
DUM = 640
NPAD = N + DUM
NSEG = 4
CSEG = NCHUNK // NSEG
CP = 624
CPL = N - (NS - 1) * CP

_mesh = plsc.VectorSubcoreMesh(
    core_axis_name="c", subcore_axis_name="s", num_cores=NC, num_subcores=NS
)


def _zero_spmem(z2_hbm, agg_sh, s):
    @pl.when(s < NS - 1)
    def _():
        pltpu.sync_copy(z2_hbm.at[pl.ds(0, CP)], agg_sh.at[pl.ds(s * CP, CP)])
    @pl.when(s == NS - 1)
    def _():
        pltpu.sync_copy(z2_hbm, agg_sh.at[pl.ds((NS - 1) * CP, CPL)])


def _copy_out(agg_sh, agg_out, c, s):
    @pl.when(s < NS - 1)
    def _():
        pltpu.sync_copy(agg_sh.at[pl.ds(s * CP, CP)],
                        agg_out.at[c, pl.ds(s * CP, CP)])
    @pl.when(s == NS - 1)
    def _():
        pltpu.sync_copy(agg_sh.at[pl.ds((NS - 1) * CP, CPL)],
                        agg_out.at[c, pl.ds((NS - 1) * CP, CPL)])


def _sc_agg_body(h_hbm, src_hbm, dst_hbm, z2_hbm, agg_out,
                 sall, dall, rA, rB, agg_sh, gsA, gsB, ssA, ssB):
    c = lax.axis_index("c")
    s = lax.axis_index("s")
    _zero_spmem(z2_hbm, agg_sh, s)
    wid = s * NC + c
    plsc.subcore_barrier()

    def gather(j, rbuf, sem):
        return pltpu.async_copy(h_hbm.at[sall.at[j]], rbuf, sem)

    def gwait(j, rbuf, sem):
        pltpu.make_async_copy(h_hbm.at[sall.at[j]], rbuf, sem).wait()

    def seg_loop(g, carry):
        pltpu.sync_copy(src_hbm.at[wid, g], sall)
        pltpu.sync_copy(dst_hbm.at[wid, g], dall)
        gather(0, rA, gsA)

        def pair(i, c2):
            ja = 2 * i
            jb = 2 * i + 1
            gather(jb, rB, gsB)
            gwait(ja, rA, gsA)
            pltpu.async_copy(rA, agg_sh.at[dall.at[ja]], ssA, add=True)
            gwait(jb, rB, gsB)
            pltpu.async_copy(rB, agg_sh.at[dall.at[jb]], ssB, add=True)
            pltpu.make_async_copy(rA, agg_sh.at[dall.at[ja]], ssA).wait()
            gather(2 * i + 2, rA, gsA)
            pltpu.make_async_copy(rB, agg_sh.at[dall.at[jb]], ssB).wait()
            return c2

        lax.fori_loop(0, (CSEG - 1) // 2, pair, 0)
        gather(CSEG - 1, rB, gsB)
        gwait(CSEG - 2, rA, gsA)
        pltpu.sync_copy(rA, agg_sh.at[dall.at[CSEG - 2]], add=True)
        gwait(CSEG - 1, rB, gsB)
        pltpu.sync_copy(rB, agg_sh.at[dall.at[CSEG - 1]], add=True)
        return carry

    lax.fori_loop(0, NSEG, seg_loop, 0)
    plsc.subcore_barrier()
    _copy_out(agg_sh, agg_out, c, s)


def _sc_deg_body(dst_hbm, ones_hbm, z2_hbm,
                 deg_out, dall, ones_v, agg_sh, ss0, ss1, ss2, ss3):
    c = lax.axis_index("c")
    s = lax.axis_index("s")
    _zero_spmem(z2_hbm, agg_sh, s)
    pltpu.sync_copy(ones_hbm, ones_v)
    wid = s * NC + c
    pltpu.sync_copy(dst_hbm.at[wid], dall)
    plsc.subcore_barrier()

    sss = (ss0, ss1, ss2, ss3)

    def quad(i, carry):
        j = 4 * i
        for k in range(4):
            pltpu.async_copy(ones_v, agg_sh.at[dall.at[j + k]],
                             sss[k], add=True)
        for k in range(4):
            pltpu.make_async_copy(ones_v, agg_sh.at[dall.at[j + k]],
                                  sss[k]).wait()
        return carry

    lax.fori_loop(0, NCHUNK // 4, quad, 0)
    plsc.subcore_barrier()
    _copy_out(agg_sh, deg_out, c, s)


_sc_agg = pl.kernel(
    _sc_agg_body,
    out_type=jax.ShapeDtypeStruct((NC, N, H), jnp.float32),
    mesh=_mesh,
    scratch_types=[
        pltpu.VMEM((CSEG, CH), jnp.int32),
        pltpu.VMEM((CSEG, CH), jnp.int32),
        pltpu.VMEM((CH, H), jnp.float32),
        pltpu.VMEM((CH, H), jnp.float32),
        pltpu.VMEM_SHARED((NPAD, H), jnp.float32),
        pltpu.SemaphoreType.DMA,
        pltpu.SemaphoreType.DMA,
        pltpu.SemaphoreType.DMA,
        pltpu.SemaphoreType.DMA,
    ],
)

_sc_deg = pl.kernel(
    _sc_deg_body,
    out_type=jax.ShapeDtypeStruct((NC, N, H), jnp.float32),
    mesh=_mesh,
    scratch_types=[
        pltpu.VMEM((NCHUNK, CH), jnp.int32),
        pltpu.VMEM((CH, H), jnp.float32),
        pltpu.VMEM_SHARED((NPAD, H), jnp.float32),
        pltpu.SemaphoreType.DMA,
        pltpu.SemaphoreType.DMA,
        pltpu.SemaphoreType.DMA,
        pltpu.SemaphoreType.DMA,
    ],
)


RB = 1000


def _proj_body(x_ref, w_ref, b_ref, o_ref):
    o_ref[...] = jnp.maximum(
        jnp.dot(x_ref[...], w_ref[...], preferred_element_type=jnp.float32)
        + b_ref[...], 0.0)


_proj = pl.pallas_call(
    _proj_body,
    grid=(N // RB,),
    in_specs=[
        pl.BlockSpec((RB, D), lambda i: (i, 0)),
        pl.BlockSpec((D, H), lambda i: (0, 0)),
        pl.BlockSpec((1, H), lambda i: (0, 0)),
    ],
    out_specs=pl.BlockSpec((RB, H), lambda i: (i, 0)),
    out_shape=jax.ShapeDtypeStruct((N, H), jnp.float32),
)


def _layer_body(p_ref, d_ref, h_ref, wl_ref, wr_ref, bb_ref, g_ref, be_ref,
                o_ref):
    p = p_ref[...]
    d = d_ref[...]
    deg = jnp.maximum(d[0, :, :1] + d[1, :, :1], 1.0)
    agg = (p[0] + p[1]) / deg
    hh = h_ref[...]
    y = (jnp.dot(agg, wl_ref[...], preferred_element_type=jnp.float32)
         + jnp.dot(hh, wr_ref[...], preferred_element_type=jnp.float32)
         + bb_ref[...])
    mu = jnp.mean(y, axis=-1, keepdims=True)
    var = jnp.mean((y - mu) ** 2, axis=-1, keepdims=True)
    y = (y - mu) / jnp.sqrt(var + 1e-5) * g_ref[...] + be_ref[...]
    y = jnp.where(y > 0, y, jnp.exp(jnp.minimum(y, 0.0)) - 1.0)
    o_ref[...] = y + hh


_layer = pl.pallas_call(
    _layer_body,
    grid=(N // RB,),
    in_specs=[
        pl.BlockSpec((NC, RB, H), lambda i: (0, i, 0)),
        pl.BlockSpec((NC, RB, H), lambda i: (0, i, 0)),
        pl.BlockSpec((RB, H), lambda i: (i, 0)),
        pl.BlockSpec((H, H), lambda i: (0, 0)),
        pl.BlockSpec((H, H), lambda i: (0, 0)),
        pl.BlockSpec((1, H), lambda i: (0, 0)),
        pl.BlockSpec((1, H), lambda i: (0, 0)),
        pl.BlockSpec((1, H), lambda i: (0, 0)),
    ],
    out_specs=pl.BlockSpec((RB, H), lambda i: (i, 0)),
    out_shape=jax.ShapeDtypeStruct((N, H), jnp.float32),
)


def _pool_body(h_ref, batch_ref, wg1_ref, bg1_ref, wg2_ref, bg2_ref,
               wc1_ref, bc1_ref, wc2_ref, bc2_ref, o_ref):
    h = h_ref[...]
    onehot = (batch_ref[...] ==
              lax.broadcasted_iota(jnp.int32, (1, B), 1)).astype(jnp.float32)
    g1 = jnp.maximum(
        jnp.dot(h, wg1_ref[...], preferred_element_type=jnp.float32)
        + bg1_ref[...], 0.0)
    gate = (jnp.dot(g1, wg2_ref[...], preferred_element_type=jnp.float32)
            + bg2_ref[...])
    m = jnp.max(jnp.where(onehot > 0, gate, -1e30), axis=0, keepdims=True)
    mb = jnp.sum(onehot * m, axis=1, keepdims=True)
    e = jnp.exp(gate - mb)
    dn = (((0,), (0,)), ((), ()))
    ssum = lax.dot_general(onehot, e, dn,
                           preferred_element_type=jnp.float32)
    pnum = lax.dot_general(onehot, e * h, dn,
                           preferred_element_type=jnp.float32)
    pooled = pnum / (ssum + 1e-16)
    z = jnp.maximum(
        jnp.dot(pooled, wc1_ref[...], preferred_element_type=jnp.float32)
        + bc1_ref[...], 0.0)
    o_ref[...] = (jnp.dot(z, wc2_ref[...], preferred_element_type=jnp.float32)
                  + bc2_ref[...])


_pool = pl.pallas_call(
    _pool_body,
    in_specs=[
        pl.BlockSpec((N, H), lambda: (0, 0)),
        pl.BlockSpec((N, 1), lambda: (0, 0)),
        pl.BlockSpec((H, H // 2), lambda: (0, 0)),
        pl.BlockSpec((1, H // 2), lambda: (0, 0)),
        pl.BlockSpec((H // 2, 1), lambda: (0, 0)),
        pl.BlockSpec((1, 1), lambda: (0, 0)),
        pl.BlockSpec((H, H // 2), lambda: (0, 0)),
        pl.BlockSpec((1, H // 2), lambda: (0, 0)),
        pl.BlockSpec((H // 2, C), lambda: (0, 0)),
        pl.BlockSpec((1, C), lambda: (0, 0)),
    ],
    out_specs=pl.BlockSpec((B, C), lambda: (0, 0)),
    out_shape=jax.ShapeDtypeStruct((B, C), jnp.float32),
)


def kernel(x, edge_index, batch, Wp, bp, Wl, bl, Wr, br, gamma, beta,
           Wg1, bg1, Wg2, bg2, Wc1, bc1, Wc2, bc2):
    npad = NW * EPP - E
    srcp = jnp.concatenate([edge_index[0], jnp.zeros((npad,), jnp.int32)])
    dstp = jnp.concatenate(
        [edge_index[1],
         N + (jnp.arange(npad, dtype=jnp.int32) % DUM)])
    src = srcp.reshape(NW, NSEG, CSEG, CH)
    dst = dstp.reshape(NW, NSEG, CSEG, CH)
    dst3 = dstp.reshape(NW, NCHUNK, CH)
    z2 = jnp.zeros((CPL, H), jnp.float32)
    ones2 = jnp.ones((CH, H), jnp.float32)

    h = _proj(x, Wp, bp.reshape(1, H))
    deg = _sc_deg(dst3, ones2, z2)

    for l in range(L):
        parts = _sc_agg(h, src, dst, z2)
        bb = (bl[l] + br[l]).reshape(1, H)
        h = _layer(parts, deg, h, Wl[l], Wr[l], bb,
                   gamma[l].reshape(1, H), beta[l].reshape(1, H))

    return _pool(h, batch.reshape(N, 1), Wg1, bg1.reshape(1, H // 2),
                 Wg2, bg2.reshape(1, 1), Wc1, bc1.reshape(1, H // 2),
                 Wc2, bc2.reshape(1, C))


# speedup vs baseline: 2.7049x; 2.7049x over previous
import jax
import jax.numpy as jnp
from jax import lax
from jax.experimental import pallas as pl
from jax.experimental.pallas import tpu as pltpu
from jax.experimental.pallas import tpu_sc as plsc

N, E, D, H, B, L, C = 10000, 320000, 128, 128, 32, 3, 6
NC, NS = 2, 16
NW = NC * NS
EPT = E // NW
CH = 80
NCHUNK = EPT // CH
NSEG = 5
CSEG = NCHUNK // NSEG
CP = 624
CPL = N - (NS - 1) * CP

_mesh = plsc.VectorSubcoreMesh(
    core_axis_name="c", subcore_axis_name="s", num_cores=NC, num_subcores=NS
)


def _zero_spmem(z2_hbm, agg_sh, s):
    @pl.when(s < NS - 1)
    def _():
        pltpu.sync_copy(z2_hbm.at[pl.ds(0, CP)], agg_sh.at[pl.ds(s * CP, CP)])
    @pl.when(s == NS - 1)
    def _():
        pltpu.sync_copy(z2_hbm, agg_sh.at[pl.ds((NS - 1) * CP, CPL)])


def _copy_out(agg_sh, agg_out, c, s):
    @pl.when(s < NS - 1)
    def _():
        pltpu.sync_copy(agg_sh.at[pl.ds(s * CP, CP)],
                        agg_out.at[c, pl.ds(s * CP, CP)])
    @pl.when(s == NS - 1)
    def _():
        pltpu.sync_copy(agg_sh.at[pl.ds((NS - 1) * CP, CPL)],
                        agg_out.at[c, pl.ds((NS - 1) * CP, CPL)])


def _sc_agg_body(h_hbm, src_hbm, dst_hbm, z2_hbm, agg_out,
                 sall, dall, r0, r1, r2, r3, agg_sh,
                 gs0, gs1, gs2, gs3, ss0, ss1, ss2, ss3):
    c = lax.axis_index("c")
    s = lax.axis_index("s")
    _zero_spmem(z2_hbm, agg_sh, s)
    wid = s * NC + c
    plsc.subcore_barrier()

    rbufs = (r0, r1, r2, r3)
    gss = (gs0, gs1, gs2, gs3)
    sss = (ss0, ss1, ss2, ss3)

    def gather(j, rbuf, sem):
        return pltpu.async_copy(h_hbm.at[sall.at[j]], rbuf, sem)

    def gwait(j, rbuf, sem):
        pltpu.make_async_copy(h_hbm.at[sall.at[j]], rbuf, sem).wait()

    def seg_loop(g, carry):
        pltpu.sync_copy(src_hbm.at[wid, g], sall)
        pltpu.sync_copy(dst_hbm.at[wid, g], dall)

        def quad(i, c2):
            j = 4 * i
            for k in range(4):
                gather(j + k, rbufs[k], gss[k])
            for k in range(4):
                gwait(j + k, rbufs[k], gss[k])
                pltpu.async_copy(
                    rbufs[k], agg_sh.at[dall.at[j + k]], sss[k], add=True)
            for k in range(4):
                pltpu.make_async_copy(
                    rbufs[k], agg_sh.at[dall.at[j + k]], sss[k]).wait()
            return c2

        lax.fori_loop(0, CSEG // 4, quad, 0)
        je = CSEG - 1
        gather(je, r0, gs0)
        gwait(je, r0, gs0)
        pltpu.sync_copy(r0, agg_sh.at[dall.at[je]], add=True)
        return carry

    lax.fori_loop(0, NSEG, seg_loop, 0)
    plsc.subcore_barrier()
    _copy_out(agg_sh, agg_out, c, s)


def _sc_deg_body(dst_hbm, ones_hbm, z2_hbm,
                 deg_out, dall, ones_v, agg_sh, ss0, ss1, ss2, ss3):
    c = lax.axis_index("c")
    s = lax.axis_index("s")
    _zero_spmem(z2_hbm, agg_sh, s)
    pltpu.sync_copy(ones_hbm, ones_v)
    wid = s * NC + c
    pltpu.sync_copy(dst_hbm.at[wid], dall)
    plsc.subcore_barrier()

    sss = (ss0, ss1, ss2, ss3)

    def quad(i, carry):
        j = 4 * i
        for k in range(4):
            pltpu.async_copy(ones_v, agg_sh.at[dall.at[j + k]],
                             sss[k], add=True)
        for k in range(4):
            pltpu.make_async_copy(ones_v, agg_sh.at[dall.at[j + k]],
                                  sss[k]).wait()
        return carry

    lax.fori_loop(0, NCHUNK // 4, quad, 0)
    pltpu.sync_copy(ones_v, agg_sh.at[dall.at[NCHUNK - 1]], add=True)
    plsc.subcore_barrier()
    _copy_out(agg_sh, deg_out, c, s)


_sc_agg = pl.kernel(
    _sc_agg_body,
    out_type=jax.ShapeDtypeStruct((NC, N, H), jnp.float32),
    mesh=_mesh,
    scratch_types=[
        pltpu.VMEM((CSEG, CH), jnp.int32),
        pltpu.VMEM((CSEG, CH), jnp.int32),
        pltpu.VMEM((CH, H), jnp.float32),
        pltpu.VMEM((CH, H), jnp.float32),
        pltpu.VMEM((CH, H), jnp.float32),
        pltpu.VMEM((CH, H), jnp.float32),
        pltpu.VMEM_SHARED((N, H), jnp.float32),
        pltpu.SemaphoreType.DMA,
        pltpu.SemaphoreType.DMA,
        pltpu.SemaphoreType.DMA,
        pltpu.SemaphoreType.DMA,
        pltpu.SemaphoreType.DMA,
        pltpu.SemaphoreType.DMA,
        pltpu.SemaphoreType.DMA,
        pltpu.SemaphoreType.DMA,
    ],
)

_sc_deg = pl.kernel(
    _sc_deg_body,
    out_type=jax.ShapeDtypeStruct((NC, N, H), jnp.float32),
    mesh=_mesh,
    scratch_types=[
        pltpu.VMEM((NCHUNK, CH), jnp.int32),
        pltpu.VMEM((CH, H), jnp.float32),
        pltpu.VMEM_SHARED((N, H), jnp.float32),
        pltpu.SemaphoreType.DMA,
        pltpu.SemaphoreType.DMA,
        pltpu.SemaphoreType.DMA,
        pltpu.SemaphoreType.DMA,
    ],
)


RB = 1000


def _proj_body(x_ref, w_ref, b_ref, o_ref):
    o_ref[...] = jnp.maximum(
        jnp.dot(x_ref[...], w_ref[...], preferred_element_type=jnp.float32)
        + b_ref[...], 0.0)


_proj = pl.pallas_call(
    _proj_body,
    grid=(N // RB,),
    in_specs=[
        pl.BlockSpec((RB, D), lambda i: (i, 0)),
        pl.BlockSpec((D, H), lambda i: (0, 0)),
        pl.BlockSpec((1, H), lambda i: (0, 0)),
    ],
    out_specs=pl.BlockSpec((RB, H), lambda i: (i, 0)),
    out_shape=jax.ShapeDtypeStruct((N, H), jnp.float32),
)


def _layer_body(p_ref, d_ref, h_ref, wl_ref, wr_ref, bb_ref, g_ref, be_ref,
                o_ref):
    p = p_ref[...]
    d = d_ref[...]
    deg = jnp.maximum(d[0, :, :1] + d[1, :, :1], 1.0)
    agg = (p[0] + p[1]) / deg
    hh = h_ref[...]
    y = (jnp.dot(agg, wl_ref[...], preferred_element_type=jnp.float32)
         + jnp.dot(hh, wr_ref[...], preferred_element_type=jnp.float32)
         + bb_ref[...])
    mu = jnp.mean(y, axis=-1, keepdims=True)
    var = jnp.mean((y - mu) ** 2, axis=-1, keepdims=True)
    y = (y - mu) / jnp.sqrt(var + 1e-5) * g_ref[...] + be_ref[...]
    y = jnp.where(y > 0, y, jnp.exp(jnp.minimum(y, 0.0)) - 1.0)
    o_ref[...] = y + hh


_layer = pl.pallas_call(
    _layer_body,
    grid=(N // RB,),
    in_specs=[
        pl.BlockSpec((NC, RB, H), lambda i: (0, i, 0)),
        pl.BlockSpec((NC, RB, H), lambda i: (0, i, 0)),
        pl.BlockSpec((RB, H), lambda i: (i, 0)),
        pl.BlockSpec((H, H), lambda i: (0, 0)),
        pl.BlockSpec((H, H), lambda i: (0, 0)),
        pl.BlockSpec((1, H), lambda i: (0, 0)),
        pl.BlockSpec((1, H), lambda i: (0, 0)),
        pl.BlockSpec((1, H), lambda i: (0, 0)),
    ],
    out_specs=pl.BlockSpec((RB, H), lambda i: (i, 0)),
    out_shape=jax.ShapeDtypeStruct((N, H), jnp.float32),
)


def _pool_body(h_ref, batch_ref, wg1_ref, bg1_ref, wg2_ref, bg2_ref,
               wc1_ref, bc1_ref, wc2_ref, bc2_ref, o_ref):
    h = h_ref[...]
    onehot = (batch_ref[...] ==
              lax.broadcasted_iota(jnp.int32, (1, B), 1)).astype(jnp.float32)
    g1 = jnp.maximum(
        jnp.dot(h, wg1_ref[...], preferred_element_type=jnp.float32)
        + bg1_ref[...], 0.0)
    gate = (jnp.dot(g1, wg2_ref[...], preferred_element_type=jnp.float32)
            + bg2_ref[...])
    m = jnp.max(jnp.where(onehot > 0, gate, -1e30), axis=0, keepdims=True)
    mb = jnp.sum(onehot * m, axis=1, keepdims=True)
    e = jnp.exp(gate - mb)
    dn = (((0,), (0,)), ((), ()))
    ssum = lax.dot_general(onehot, e, dn,
                           preferred_element_type=jnp.float32)
    pnum = lax.dot_general(onehot, e * h, dn,
                           preferred_element_type=jnp.float32)
    pooled = pnum / (ssum + 1e-16)
    z = jnp.maximum(
        jnp.dot(pooled, wc1_ref[...], preferred_element_type=jnp.float32)
        + bc1_ref[...], 0.0)
    o_ref[...] = (jnp.dot(z, wc2_ref[...], preferred_element_type=jnp.float32)
                  + bc2_ref[...])


_pool = pl.pallas_call(
    _pool_body,
    in_specs=[
        pl.BlockSpec((N, H), lambda: (0, 0)),
        pl.BlockSpec((N, 1), lambda: (0, 0)),
        pl.BlockSpec((H, H // 2), lambda: (0, 0)),
        pl.BlockSpec((1, H // 2), lambda: (0, 0)),
        pl.BlockSpec((H // 2, 1), lambda: (0, 0)),
        pl.BlockSpec((1, 1), lambda: (0, 0)),
        pl.BlockSpec((H, H // 2), lambda: (0, 0)),
        pl.BlockSpec((1, H // 2), lambda: (0, 0)),
        pl.BlockSpec((H // 2, C), lambda: (0, 0)),
        pl.BlockSpec((1, C), lambda: (0, 0)),
    ],
    out_specs=pl.BlockSpec((B, C), lambda: (0, 0)),
    out_shape=jax.ShapeDtypeStruct((B, C), jnp.float32),
)


def kernel(x, edge_index, batch, Wp, bp, Wl, bl, Wr, br, gamma, beta,
           Wg1, bg1, Wg2, bg2, Wc1, bc1, Wc2, bc2):
    src = edge_index[0].reshape(NW, NSEG, CSEG, CH)
    dst = edge_index[1].reshape(NW, NSEG, CSEG, CH)
    dst3 = edge_index[1].reshape(NW, NCHUNK, CH)
    z2 = jnp.zeros((CPL, H), jnp.float32)
    ones2 = jnp.ones((CH, H), jnp.float32)

    h = _proj(x, Wp, bp.reshape(1, H))
    deg = _sc_deg(dst3, ones2, z2)

    for l in range(L):
        parts = _sc_agg(h, src, dst, z2)
        bb = (bl[l] + br[l]).reshape(1, H)
        h = _layer(parts, deg, h, Wl[l], Wr[l], bb,
                   gamma[l].reshape(1, H), beta[l].reshape(1, H))

    return _pool(h, batch.reshape(N, 1), Wg1, bg1.reshape(1, H // 2),
                 Wg2, bg2.reshape(1, 1), Wc1, bc1.reshape(1, H // 2),
                 Wc2, bc2.reshape(1, C))
